# baseline (speedup 1.0000x reference)
import functools

import jax
import jax.numpy as jnp
from jax import lax
from jax.experimental import pallas as pl
from jax.experimental.pallas import tpu as pltpu
from jax.experimental.pallas import tpu_sc as plsc

EMB = 64
BATCH = 4096
L = 16
NC = 2
NS = 16
NW = NC * NS
BPW = BATCH // NW
NG = BPW // L


def _sc_gather_dot(u2, f2, ub_t, fb_t, uid2, fid2, uoff, foff):
    mesh = plsc.VectorSubcoreMesh(core_axis_name="c", subcore_axis_name="s")

    @functools.partial(
        pl.kernel,
        mesh=mesh,
        compiler_params=pltpu.CompilerParams(needs_layout_passes=False),
        out_type=(
            jax.ShapeDtypeStruct((NW, L), jnp.float32),
            jax.ShapeDtypeStruct((BATCH,), jnp.float32),
            jax.ShapeDtypeStruct((BATCH,), jnp.float32),
        ),
        scratch_types=[
            pltpu.VMEM((BPW,), jnp.int32),
            pltpu.VMEM((BPW,), jnp.int32),
            pltpu.VMEM((BPW,), jnp.int32),
            pltpu.VMEM((BPW,), jnp.int32),
            pltpu.VMEM((BPW, 2 * EMB), jnp.float32),
            pltpu.VMEM((BPW, 2 * EMB), jnp.float32),
            pltpu.VMEM((BPW,), jnp.float32),
            pltpu.VMEM((BPW,), jnp.float32),
            pltpu.VMEM((L,), jnp.float32),
            pltpu.SemaphoreType.DMA,
            pltpu.SemaphoreType.DMA,
            pltpu.SemaphoreType.DMA,
            pltpu.SemaphoreType.DMA,
        ],
    )
    def k(u2_h, f2_h, ub_h, fb_h, uid2_h, fid2_h, uoff_h, foff_h,
          part_out, ub_out, fb_out,
          uidx_v, fidx_v, uoff_v, foff_v, upairs_v, fpairs_v, ub_v, fb_v,
          acc_v, sem_u, sem_f, sem_ub, sem_fb):
        wid = lax.axis_index("s") * NC + lax.axis_index("c")
        base = wid * BPW
        pltpu.sync_copy(uid2_h.at[pl.ds(base, BPW)], uidx_v)
        pltpu.sync_copy(fid2_h.at[pl.ds(base, BPW)], fidx_v)
        pltpu.sync_copy(uoff_h.at[pl.ds(base, BPW)], uoff_v)
        pltpu.sync_copy(foff_h.at[pl.ds(base, BPW)], foff_v)
        cu = pltpu.async_copy(u2_h.at[uidx_v], upairs_v, sem_u)
        cf = pltpu.async_copy(f2_h.at[fidx_v], fpairs_v, sem_f)
        cub = pltpu.async_copy(ub_h.at[uidx_v], ub_v, sem_ub)
        cfb = pltpu.async_copy(fb_h.at[fidx_v], fb_v, sem_fb)
        cu.wait()
        cf.wait()

        lane = jnp.arange(L, dtype=jnp.int32)

        def group(g, accs):
            rows = lane + g * L
            uo = uoff_v[pl.ds(g * L, L)]
            fo = foff_v[pl.ds(g * L, L)]

            def step(d, accs2):
                a0, a1, a2, a3 = accs2
                d0 = d * 4
                a0 = a0 + (plsc.load_gather(upairs_v, [rows, uo + d0]) *
                           plsc.load_gather(fpairs_v, [rows, fo + d0]))
                a1 = a1 + (plsc.load_gather(upairs_v, [rows, uo + (d0 + 1)]) *
                           plsc.load_gather(fpairs_v, [rows, fo + (d0 + 1)]))
                a2 = a2 + (plsc.load_gather(upairs_v, [rows, uo + (d0 + 2)]) *
                           plsc.load_gather(fpairs_v, [rows, fo + (d0 + 2)]))
                a3 = a3 + (plsc.load_gather(upairs_v, [rows, uo + (d0 + 3)]) *
                           plsc.load_gather(fpairs_v, [rows, fo + (d0 + 3)]))
                return (a0, a1, a2, a3)

            return lax.fori_loop(0, EMB // 4, step, accs)

        z = jnp.zeros((L,), jnp.float32)
        a0, a1, a2, a3 = lax.fori_loop(0, NG, group, (z, z, z, z))
        acc_v[...] = (a0 + a1) + (a2 + a3)
        pltpu.sync_copy(acc_v, part_out.at[wid])
        cub.wait()
        cfb.wait()
        pltpu.sync_copy(ub_v, ub_out.at[pl.ds(base, BPW)])
        pltpu.sync_copy(fb_v, fb_out.at[pl.ds(base, BPW)])

    return k(u2, f2, ub_t, fb_t, uid2, fid2, uoff, foff)


def _tc_mlp(partials, ub, fb, w1r, b1r, w2, b2r, w3r, b3r):
    def body(p_ref, ub_ref, fb_ref, w1_ref, b1_ref, w2_ref, b2_ref,
             w3_ref, b3_ref, out_ref):
        s = jnp.sum(p_ref[...])
        x = s + ub_ref[...] + fb_ref[...]
        h1 = jnp.maximum(x * w1_ref[...] + b1_ref[...], 0.0)
        h2 = jnp.maximum(
            jnp.dot(h1, w2_ref[...], preferred_element_type=jnp.float32)
            + b2_ref[...], 0.0)
        zz = jnp.sum(h2 * w3_ref[...], axis=1, keepdims=True) + b3_ref[...]
        out_ref[...] = 1.0 / (1.0 + jnp.exp(-zz))

    return pl.pallas_call(
        body,
        out_shape=jax.ShapeDtypeStruct((BATCH, 1), jnp.float32),
    )(partials, ub, fb, w1r, b1r, w2, b2r, w3r, b3r)


def kernel(inputs, user_emb, user_bias, food_emb, food_bias, W1, b1, W2, b2, W3, b3):
    idx = inputs.astype(jnp.int32)
    uid = idx[:, 0]
    fid = idx[:, 1]
    uid2 = uid >> 1
    fid2 = fid >> 1
    uoff = (uid & 1) * EMB
    foff = (fid & 1) * EMB
    u2 = user_emb.reshape(-1, 2 * EMB)
    f2 = food_emb.reshape(-1, 2 * EMB)
    partials, ub, fb = _sc_gather_dot(
        u2, f2, user_bias.reshape(-1), food_bias.reshape(-1),
        uid2, fid2, uoff, foff)
    return _tc_mlp(
        partials, ub.reshape(BATCH, 1), fb.reshape(BATCH, 1),
        W1.reshape(1, 128), b1.reshape(1, 128),
        W2, b2.reshape(1, 64),
        W3.reshape(1, 64), b3.reshape(1, 1))


# speedup vs baseline: 2.3191x; 2.3191x over previous
import functools

import jax
import jax.numpy as jnp
from jax import lax
from jax.experimental import pallas as pl
from jax.experimental.pallas import tpu as pltpu
from jax.experimental.pallas import tpu_sc as plsc

EMB = 64
BATCH = 4096
NROWS = 100000
L = 16
NC = 2
NS = 16
NW = NC * NS
NCHUNK = BATCH // L


def _sc_gather_dot(u_t, f_t, ub1, fb1, uid, fid):
    mesh = plsc.VectorSubcoreMesh(core_axis_name="c", subcore_axis_name="s")

    @functools.partial(
        pl.kernel,
        mesh=mesh,
        compiler_params=pltpu.CompilerParams(needs_layout_passes=False),
        out_type=(
            jax.ShapeDtypeStruct((NW, L), jnp.float32),
            jax.ShapeDtypeStruct((BATCH,), jnp.float32),
            jax.ShapeDtypeStruct((BATCH,), jnp.float32),
        ),
        scratch_types=[
            pltpu.VMEM((BATCH,), jnp.int32),
            pltpu.VMEM((BATCH,), jnp.int32),
            pltpu.VMEM((NROWS,), jnp.float32),
            pltpu.VMEM((BATCH,), jnp.float32),
            pltpu.VMEM((L,), jnp.float32),
        ],
    )
    def k(ut_h, ft_h, ub_h, fb_h, uid_h, fid_h,
          part_out, ubg_out, fbg_out,
          uid_v, fid_v, row_v, g_v, acc_v):
        wid = lax.axis_index("s") * NC + lax.axis_index("c")
        pltpu.sync_copy(uid_h, uid_v)
        pltpu.sync_copy(fid_h, fid_v)

        def gather_to_g(c, _):
            for q in range(4):
                sl = pl.ds((c * 4 + q) * L, L)
                g_v[sl] = plsc.load_gather(row_v, [uid_v[sl]])
            return 0

        def gather_f_fma(c, accs):
            a = list(accs)
            for q in range(4):
                sl = pl.ds((c * 4 + q) * L, L)
                a[q] = a[q] + plsc.load_gather(row_v, [fid_v[sl]]) * g_v[sl]
            return tuple(a)

        z = jnp.zeros((L,), jnp.float32)
        accs = (z, z, z, z)
        for p in range(2):
            d = wid + NW * p
            pltpu.sync_copy(ut_h.at[d], row_v)
            lax.fori_loop(0, NCHUNK // 4, gather_to_g, 0)
            pltpu.sync_copy(ft_h.at[d], row_v)
            accs = lax.fori_loop(0, NCHUNK // 4, gather_f_fma, accs)

        a0, a1, a2, a3 = accs
        acc_v[...] = (a0 + a1) + (a2 + a3)
        pltpu.sync_copy(acc_v, part_out.at[wid])

        @pl.when(wid == 0)
        def _():
            pltpu.sync_copy(ub_h, row_v)
            lax.fori_loop(0, NCHUNK // 4, gather_to_g, 0)
            pltpu.sync_copy(g_v, ubg_out)

        @pl.when(wid == 1)
        def _():
            pltpu.sync_copy(fb_h, row_v)

            def gather_fb(c, _):
                for q in range(4):
                    sl = pl.ds((c * 4 + q) * L, L)
                    g_v[sl] = plsc.load_gather(row_v, [fid_v[sl]])
                return 0

            lax.fori_loop(0, NCHUNK // 4, gather_fb, 0)
            pltpu.sync_copy(g_v, fbg_out)

    return k(u_t, f_t, ub1, fb1, uid, fid)


def _tc_mlp(partials, ub, fb, w1r, b1r, w2, b2r, w3r, b3r):
    def body(p_ref, ub_ref, fb_ref, w1_ref, b1_ref, w2_ref, b2_ref,
             w3_ref, b3_ref, out_ref):
        s = jnp.sum(p_ref[...])
        x = s + ub_ref[...] + fb_ref[...]
        h1 = jnp.maximum(x * w1_ref[...] + b1_ref[...], 0.0)
        h2 = jnp.maximum(
            jnp.dot(h1, w2_ref[...], preferred_element_type=jnp.float32)
            + b2_ref[...], 0.0)
        zz = jnp.sum(h2 * w3_ref[...], axis=1, keepdims=True) + b3_ref[...]
        out_ref[...] = 1.0 / (1.0 + jnp.exp(-zz))

    return pl.pallas_call(
        body,
        out_shape=jax.ShapeDtypeStruct((BATCH, 1), jnp.float32),
    )(partials, ub, fb, w1r, b1r, w2, b2r, w3r, b3r)


def kernel(inputs, user_emb, user_bias, food_emb, food_bias, W1, b1, W2, b2, W3, b3):
    idx = inputs.astype(jnp.int32)
    uid = idx[:, 0]
    fid = idx[:, 1]
    partials, ubg, fbg = _sc_gather_dot(
        user_emb.T, food_emb.T,
        user_bias.reshape(-1), food_bias.reshape(-1),
        uid, fid)
    return _tc_mlp(
        partials, ubg.reshape(BATCH, 1), fbg.reshape(BATCH, 1),
        W1.reshape(1, 128), b1.reshape(1, 128),
        W2, b2.reshape(1, 64),
        W3.reshape(1, 64), b3.reshape(1, 1))
